# baseline (speedup 1.0000x reference)
import functools

import jax
import jax.numpy as jnp
from jax import lax
from jax.experimental import pallas as pl
from jax.experimental.pallas import tpu as pltpu
from jax.experimental.pallas import tpu_sc as plsc

_B = 16384
_NC = 2
_NS = 16
_NW = _NC * _NS
_BPW = _B // _NW
_L = 16

_mesh = plsc.VectorSubcoreMesh(core_axis_name="c", subcore_axis_name="s")


@functools.partial(
    pl.kernel,
    mesh=_mesh,
    out_type=jax.ShapeDtypeStruct((_B,), jnp.float32),
    scratch_types=[
        pltpu.VMEM((_BPW,), jnp.int32),
        pltpu.VMEM((_BPW,), jnp.int32),
        pltpu.VMEM((_BPW,), jnp.float32),
        pltpu.VMEM((_L,), jnp.float32),
    ],
)
def _tab_disc(a0_hbm, a1_hbm, logits_hbm, out_hbm, a0_v, a1_v, out_v, tab_v):
    wid = lax.axis_index("s") * _NC + lax.axis_index("c")
    base = wid * _BPW
    pltpu.sync_copy(logits_hbm, tab_v)
    pltpu.sync_copy(a0_hbm.at[pl.ds(base, _BPW)], a0_v)
    pltpu.sync_copy(a1_hbm.at[pl.ds(base, _BPW)], a1_v)
    logits_vec = tab_v[...]
    sig = 1.0 / (1.0 + jnp.exp(-logits_vec))

    def body(i, carry):
        off = i * _L
        a0 = a0_v[pl.ds(off, _L)]
        a1 = a1_v[pl.ds(off, _L)]
        idx = jnp.clip(a0 * 2 + a1, 0, 3)
        out_v[pl.ds(off, _L)] = sig.at[idx].get(mode="promise_in_bounds")
        return carry

    lax.fori_loop(0, _BPW // _L, body, 0)
    pltpu.sync_copy(out_v, out_hbm.at[pl.ds(base, _BPW)])


def kernel(a0, a1, logits):
    logits16 = jnp.zeros((_L,), jnp.float32).at[:4].set(logits.astype(jnp.float32))
    return _tab_disc(a0.astype(jnp.int32), a1.astype(jnp.int32), logits16)
